# baseline (speedup 1.0000x reference)
import jax
import jax.numpy as jnp
from jax import lax
from jax.experimental import pallas as pl
from jax.experimental.pallas import tpu as pltpu
from jax.experimental.pallas import tpu_sc as plsc

_NC = 2
_NS = 16
_NW = _NC * _NS
_CHUNK = 128
_D = 64
_DT = _D // 8
_NBUF = 4
_AHEAD = 2


def _body(x_hbm, table_hbm, out_hbm, idx_v, rows_v, tile_v, gsem, ssem):
    nchunk = x_hbm.shape[1]
    n_s = out_hbm.shape[0]
    wid = lax.axis_index("s") * _NC + lax.axis_index("c")
    pltpu.sync_copy(x_hbm.at[wid], idx_v)
    iota = lax.broadcasted_iota(jnp.int32, (16,), 0)

    def g_desc(j, b):
        return pltpu.make_async_copy(
            table_hbm.at[idx_v.at[j]], rows_v.at[b], gsem)

    def s_desc(j, b):
        t = j // n_s
        s = j - t * n_s
        return pltpu.make_async_copy(
            tile_v.at[b], out_hbm.at[s, :, wid * (nchunk // n_s) + t], ssem)

    for j in range(_AHEAD):
        g_desc(j, j % _NBUF).start()

    def transpose_chunk(b):
        rows_b = rows_v.at[b]
        tile_b = tile_v.at[b]

        def per_dtile(dt, carry):
            for ds in range(8):
                d = dt * 8 + ds
                col = jnp.full((16,), d, jnp.int32)
                for k in range(8):
                    v = plsc.load_gather(rows_b, [iota + 16 * k, col])
                    tile_b[dt, ds, pl.ds(16 * k, 16)] = v
            return carry

        lax.fori_loop(0, _DT, per_dtile, 0)

    def group(g, carry):
        for b in range(_NBUF):
            j = g * _NBUF + b

            @pl.when(j + _AHEAD < nchunk)
            def _():
                g_desc(j + _AHEAD, (b + _AHEAD) % _NBUF).start()

            g_desc(j, b).wait()

            @pl.when(g >= 1)
            def _():
                s_desc(j - _NBUF, b).wait()

            transpose_chunk(b)
            s_desc(j, b).start()
        return carry

    lax.fori_loop(0, nchunk // _NBUF, group, 0)
    for j in range(nchunk - _NBUF, nchunk):
        s_desc(j, j % _NBUF).wait()


def kernel(x, table):
    bsz, n_s = x.shape
    nbt = bsz // _CHUNK
    tpw = nbt // _NW
    nchunk = tpw * n_s
    xt = (x.reshape(_NW, tpw, _CHUNK, n_s)
          .transpose(0, 1, 3, 2)
          .reshape(_NW, nchunk, _CHUNK)
          .astype(jnp.int32))
    mesh = plsc.VectorSubcoreMesh(core_axis_name="c", subcore_axis_name="s")
    out5 = pl.kernel(
        _body,
        out_type=jax.ShapeDtypeStruct((n_s, _DT, nbt, 8, _CHUNK), jnp.float32),
        mesh=mesh,
        scratch_types=[
            pltpu.VMEM((nchunk, _CHUNK), jnp.int32),
            pltpu.VMEM((_NBUF, _CHUNK, _D), jnp.float32),
            pltpu.VMEM((_NBUF, _DT, 8, _CHUNK), jnp.float32),
            pltpu.SemaphoreType.DMA,
            pltpu.SemaphoreType.DMA,
        ],
        compiler_params=pltpu.CompilerParams(
            use_tc_tiling_on_sc=False, needs_layout_passes=False),
    )(xt, table)
    return out5.transpose(2, 4, 0, 1, 3).reshape(bsz, n_s, _D)


# speedup vs baseline: 1.4499x; 1.4499x over previous
import jax
import jax.numpy as jnp
from jax import lax
from jax.experimental import pallas as pl
from jax.experimental.pallas import tpu as pltpu
from jax.experimental.pallas import tpu_sc as plsc

_NC = 2
_NS = 16
_NW = _NC * _NS
_CHUNK = 128
_D = 64
_DT = _D // 8
_NBUF = 4
_AHEAD = 2


def _body(x_hbm, table_hbm, out_hbm, idx_v, rows_v, tile_v, gsem, ssem):
    nchunk = x_hbm.shape[1]
    n_s = out_hbm.shape[0]
    wid = lax.axis_index("s") * _NC + lax.axis_index("c")
    pltpu.sync_copy(x_hbm.at[wid], idx_v)
    iota = lax.broadcasted_iota(jnp.int32, (16,), 0)

    def g_desc(j, b):
        return pltpu.make_async_copy(
            table_hbm.at[idx_v.at[j]], rows_v.at[b], gsem)

    def s_desc(j, b):
        t = j // n_s
        s = j - t * n_s
        return pltpu.make_async_copy(
            tile_v.at[b], out_hbm.at[s, :, wid * (nchunk // n_s) + t], ssem)

    for j in range(_AHEAD):
        g_desc(j, j % _NBUF).start()

    def transpose_chunk(b):
        rows_b = rows_v.at[b]
        tile_b = tile_v.at[b]

        @plsc.parallel_loop(0, _D, unroll=8)
        def _(d):
            dt = d // 8
            ds = d - dt * 8
            col = jnp.full((16,), d, jnp.int32)
            for k in range(8):
                v = plsc.load_gather(rows_b, [iota + 16 * k, col])
                tile_b[dt, ds, pl.ds(16 * k, 16)] = v

    def group(g, carry):
        for b in range(_NBUF):
            j = g * _NBUF + b

            @pl.when(j + _AHEAD < nchunk)
            def _():
                g_desc(j + _AHEAD, (b + _AHEAD) % _NBUF).start()

            g_desc(j, b).wait()

            @pl.when(g >= 1)
            def _():
                s_desc(j - _NBUF, b).wait()

            transpose_chunk(b)
            s_desc(j, b).start()
        return carry

    lax.fori_loop(0, nchunk // _NBUF, group, 0)
    for j in range(nchunk - _NBUF, nchunk):
        s_desc(j, j % _NBUF).wait()


def kernel(x, table):
    bsz, n_s = x.shape
    nbt = bsz // _CHUNK
    tpw = nbt // _NW
    nchunk = tpw * n_s
    xt = (x.reshape(_NW, tpw, _CHUNK, n_s)
          .transpose(0, 1, 3, 2)
          .reshape(_NW, nchunk, _CHUNK)
          .astype(jnp.int32))
    mesh = plsc.VectorSubcoreMesh(core_axis_name="c", subcore_axis_name="s")
    out5 = pl.kernel(
        _body,
        out_type=jax.ShapeDtypeStruct((n_s, _DT, nbt, 8, _CHUNK), jnp.float32),
        mesh=mesh,
        scratch_types=[
            pltpu.VMEM((nchunk, _CHUNK), jnp.int32),
            pltpu.VMEM((_NBUF, _CHUNK, _D), jnp.float32),
            pltpu.VMEM((_NBUF, _DT, 8, _CHUNK), jnp.float32),
            pltpu.SemaphoreType.DMA,
            pltpu.SemaphoreType.DMA,
        ],
        compiler_params=pltpu.CompilerParams(
            use_tc_tiling_on_sc=False, needs_layout_passes=False),
    )(xt, table)
    return out5.transpose(2, 4, 0, 1, 3).reshape(bsz, n_s, _D)
